# Initial kernel scaffold; baseline (speedup 1.0000x reference)
import functools

import jax
import jax.numpy as jnp
from jax import lax
from jax.experimental import pallas as pl
from jax.experimental.pallas import tpu as pltpu
from jax.experimental.pallas import tpu_sc as plsc

F = 26
B = 1024
L = 200
CARD1 = 100001
EMB = 16
BL = B * L
NSLOT = F + 1
M = BL * NSLOT
NW = 32
IW = 128
ROWS_W = M // NW
GPW = ROWS_W // IW
CH = 18
STEPS = GPW // CH


def _gather_kernel(table_hbm, idx_hbm, out_hbm, idx_v, rows_v, sem):
    c = lax.axis_index("c")
    s = lax.axis_index("s")
    wid = s * 2 + c
    row0 = wid * GPW

    def step(g, carry):
        base = row0 + g * CH
        pltpu.sync_copy(idx_hbm.at[pl.ds(base, CH)], idx_v)
        copies = [
            pltpu.async_copy(
                table_hbm.at[idx_v.at[j]],
                rows_v.at[pl.ds(j * IW, IW), :],
                sem,
            )
            for j in range(CH)
        ]
        for cp in copies:
            cp.wait()
        pltpu.sync_copy(rows_v, out_hbm.at[pl.ds(base * IW, CH * IW)])
        return carry

    lax.fori_loop(0, STEPS, step, 0)


@jax.jit
def _run(big_table, idx2d):
    mesh = plsc.VectorSubcoreMesh(core_axis_name="c", subcore_axis_name="s")
    kfn = functools.partial(
        pl.kernel,
        mesh=mesh,
        out_type=jax.ShapeDtypeStruct((M, EMB), jnp.float32),
        scratch_types=[
            pltpu.VMEM((CH, IW), jnp.int32),
            pltpu.VMEM((CH * IW, EMB), jnp.float32),
            pltpu.SemaphoreType.DMA,
        ],
    )(_gather_kernel)
    return kfn(big_table, idx2d)


def kernel(indices, product, tables, product_table):
    big = jnp.concatenate(
        [tables.reshape(F * CARD1, EMB), product_table], axis=0
    )
    feat_idx = (
        indices.reshape(F, BL).T.astype(jnp.int32)
        + (jnp.arange(F, dtype=jnp.int32) * CARD1)[None, :]
    )
    prod_idx = (
        jnp.repeat(product.astype(jnp.int32), L) + F * CARD1
    )
    idx_all = jnp.concatenate([feat_idx, prod_idx[:, None]], axis=1)
    out_flat = _run(big, idx_all.reshape(M // IW, IW))
    return out_flat.reshape(B, L, NSLOT * EMB)


# speedup vs baseline: 1.3074x; 1.3074x over previous
import functools

import jax
import jax.numpy as jnp
from jax import lax
from jax.experimental import pallas as pl
from jax.experimental.pallas import tpu as pltpu
from jax.experimental.pallas import tpu_sc as plsc

F = 26
B = 1024
L = 200
CARD1 = 100001
EMB = 16
BL = B * L
NSLOT = F + 1
M = BL * NSLOT
NW = 32
IW = 128
CH = 16
CHUNK = CH * IW
NCH = M // CHUNK
STEPS = -(-NCH // NW)


def _gather_kernel(table_hbm, idx_hbm, out_hbm, idx_v, rows_v, sem):
    c = lax.axis_index("c")
    s = lax.axis_index("s")
    wid = s * 2 + c

    def step(g, carry):
        cid = g * NW + wid

        @pl.when(cid < NCH)
        def _():
            pltpu.sync_copy(idx_hbm.at[cid], idx_v)
            copies = [
                pltpu.async_copy(
                    table_hbm.at[idx_v.at[j]],
                    rows_v.at[pl.ds(j * IW, IW), :],
                    sem,
                )
                for j in range(CH)
            ]
            for cp in copies:
                cp.wait()
            pltpu.sync_copy(rows_v, out_hbm.at[pl.ds(cid * CHUNK, CHUNK)])

        return carry

    lax.fori_loop(0, STEPS, step, 0)


@jax.jit
def _run(big_table, idx2d):
    mesh = plsc.VectorSubcoreMesh(core_axis_name="c", subcore_axis_name="s")
    kfn = functools.partial(
        pl.kernel,
        mesh=mesh,
        compiler_params=pltpu.CompilerParams(use_tc_tiling_on_sc=False),
        out_type=jax.ShapeDtypeStruct((M, EMB), jnp.float32),
        scratch_types=[
            pltpu.VMEM((CH, IW), jnp.int32),
            pltpu.VMEM((CHUNK, EMB), jnp.float32),
            pltpu.SemaphoreType.DMA,
        ],
    )(_gather_kernel)
    return kfn(big_table, idx2d)


def kernel(indices, product, tables, product_table):
    big = jnp.concatenate(
        [tables.reshape(F * CARD1, EMB), product_table], axis=0
    )
    feat_idx = (
        indices.reshape(F, BL).T.astype(jnp.int32)
        + (jnp.arange(F, dtype=jnp.int32) * CARD1)[None, :]
    )
    prod_idx = (
        jnp.repeat(product.astype(jnp.int32), L) + F * CARD1
    )
    idx_all = jnp.concatenate([feat_idx, prod_idx[:, None]], axis=1)
    out_flat = _run(big, idx_all.reshape(NCH, CH, IW))
    return out_flat.reshape(B, L, NSLOT * EMB)


# speedup vs baseline: 1.3463x; 1.0298x over previous
import functools

import jax
import jax.numpy as jnp
from jax import lax
from jax.experimental import pallas as pl
from jax.experimental.pallas import tpu as pltpu
from jax.experimental.pallas import tpu_sc as plsc

F = 26
B = 1024
L = 200
CARD1 = 100001
EMB = 16
NSLOT = F + 1
NW = 32
BPW = B // NW
S0 = 128
S1 = L - S0


def _emb_kernel(idx_hbm, prod_hbm, tab_hbm, ptab_hbm, out_hbm,
                idx_v, gbuf_v, pbuf_v, pidx_v, prows_v, gsem, wsem):
    c = lax.axis_index("c")
    s = lax.axis_index("s")
    wid = s * 2 + c
    b0 = wid * BPW

    pltpu.sync_copy(prod_hbm.at[pl.ds(b0, BPW)], pidx_v)
    pltpu.async_copy(ptab_hbm.at[pidx_v], prows_v, gsem).wait()

    def drain_writes():
        pltpu.make_async_copy(
            tab_hbm.at[pl.ds(0, F * L)], gbuf_v, wsem).wait()
        pltpu.make_async_copy(
            ptab_hbm.at[pl.ds(0, L)], pbuf_v, wsem).wait()

    def do_batch(g, carry):
        b = b0 + g
        pltpu.sync_copy(idx_hbm.at[:, b, :], idx_v)

        @pl.when(g > 0)
        def _():
            drain_writes()

        def gat(f, carry2):
            pltpu.async_copy(
                tab_hbm.at[idx_v.at[f, pl.ds(0, S0)]],
                gbuf_v.at[pl.ds(f * L, S0), :],
                gsem,
            )
            pltpu.async_copy(
                tab_hbm.at[idx_v.at[f, pl.ds(S0, S1)]],
                gbuf_v.at[pl.ds(f * L + S0, S1), :],
                gsem,
            )
            return carry2

        lax.fori_loop(0, F, gat, 0)

        prow = prows_v[g, :]

        def fill(l, carry3):
            pbuf_v[l, :] = prow
            return carry3

        lax.fori_loop(0, L, fill, 0)

        pltpu.make_async_copy(
            tab_hbm.at[pl.ds(0, F * L)], gbuf_v, gsem).wait()

        def wr(f, carry4):
            pltpu.async_copy(
                gbuf_v.at[pl.ds(f * L, L), :],
                out_hbm.at[b].at[:, pl.ds(f * EMB, EMB)],
                wsem,
            )
            return carry4

        lax.fori_loop(0, F, wr, 0)
        pltpu.async_copy(
            pbuf_v, out_hbm.at[b].at[:, pl.ds(F * EMB, EMB)], wsem)
        return carry

    lax.fori_loop(0, BPW, do_batch, 0)
    drain_writes()


@jax.jit
def _run(idxg, product, tab_flat, product_table):
    mesh = plsc.VectorSubcoreMesh(core_axis_name="c", subcore_axis_name="s")
    kfn = functools.partial(
        pl.kernel,
        mesh=mesh,
        compiler_params=pltpu.CompilerParams(use_tc_tiling_on_sc=False),
        out_type=jax.ShapeDtypeStruct((B, L, NSLOT * EMB), jnp.float32),
        scratch_types=[
            pltpu.VMEM((F, L), jnp.int32),
            pltpu.VMEM((F * L, EMB), jnp.float32),
            pltpu.VMEM((L, EMB), jnp.float32),
            pltpu.VMEM((BPW,), jnp.int32),
            pltpu.VMEM((BPW, EMB), jnp.float32),
            pltpu.SemaphoreType.DMA,
            pltpu.SemaphoreType.DMA,
        ],
    )(_emb_kernel)
    return kfn(idxg, product, tab_flat, product_table)


def kernel(indices, product, tables, product_table):
    idxg = indices.astype(jnp.int32) + (
        jnp.arange(F, dtype=jnp.int32) * CARD1)[:, None, None]
    return _run(
        idxg,
        product.astype(jnp.int32),
        tables.reshape(F * CARD1, EMB),
        product_table,
    )


# speedup vs baseline: 3.5878x; 2.6650x over previous
import functools

import jax
import jax.numpy as jnp
from jax import lax
from jax.experimental import pallas as pl
from jax.experimental.pallas import tpu as pltpu
from jax.experimental.pallas import tpu_sc as plsc

F = 26
B = 1024
L = 200
CARD1 = 100001
EMB = 16
NSLOT = F + 1
NW = 32
BPW = B // NW
SLAB = F * L
S0 = 128
S1 = L - S0


def _emb_kernel(idx_hbm, prod_hbm, tab3_hbm, ptab_hbm, out_hbm,
                idx_v, gbuf_v, pbuf_v, pidx_v, prows_v, gsem, wsem):
    c = lax.axis_index("c")
    s = lax.axis_index("s")
    wid = s * 2 + c
    b0 = wid * BPW
    tab_hbm = tab3_hbm

    pltpu.sync_copy(prod_hbm.at[pl.ds(b0, BPW)], pidx_v)
    pltpu.async_copy(ptab_hbm.at[pidx_v], prows_v, gsem).wait()

    def drain_writes():
        pltpu.make_async_copy(
            tab_hbm.at[pl.ds(0, F * L)], gbuf_v, wsem).wait()
        pltpu.make_async_copy(
            ptab_hbm.at[pl.ds(0, L)], pbuf_v, wsem).wait()

    def do_batch(g, carry):
        b = b0 + g
        pltpu.sync_copy(idx_hbm.at[pl.ds(b * SLAB, SLAB)], idx_v)

        @pl.when(g > 0)
        def _():
            drain_writes()

        def gat(f, carry2):
            pltpu.async_copy(
                tab_hbm.at[idx_v.at[pl.ds(f * L, S0)]],
                gbuf_v.at[pl.ds(f * L, S0), :],
                gsem,
            )
            pltpu.async_copy(
                tab_hbm.at[idx_v.at[pl.ds(f * L + S0, S1)]],
                gbuf_v.at[pl.ds(f * L + S0, S1), :],
                gsem,
            )
            return carry2

        lax.fori_loop(0, F, gat, 0)

        prow = prows_v[g, :]

        def fill(l, carry3):
            pbuf_v[l, :] = prow
            return carry3

        lax.fori_loop(0, L, fill, 0)

        pltpu.make_async_copy(
            tab_hbm.at[pl.ds(0, F * L)], gbuf_v, gsem).wait()

        def wr(f, carry4):
            pltpu.async_copy(
                gbuf_v.at[pl.ds(f * L, L), :],
                out_hbm.at[b].at[:, pl.ds(f * EMB, EMB)],
                wsem,
            )
            return carry4

        lax.fori_loop(0, F, wr, 0)
        pltpu.async_copy(
            pbuf_v, out_hbm.at[b].at[:, pl.ds(F * EMB, EMB)], wsem)
        return carry

    lax.fori_loop(0, BPW, do_batch, 0)
    drain_writes()


@jax.jit
def _run(idx_flat, product, tables, product_table):
    mesh = plsc.VectorSubcoreMesh(core_axis_name="c", subcore_axis_name="s")
    kfn = functools.partial(
        pl.kernel,
        mesh=mesh,
        compiler_params=pltpu.CompilerParams(use_tc_tiling_on_sc=False),
        out_type=jax.ShapeDtypeStruct((B, L, NSLOT * EMB), jnp.float32),
        scratch_types=[
            pltpu.VMEM((SLAB,), jnp.int32),
            pltpu.VMEM((SLAB, EMB), jnp.float32),
            pltpu.VMEM((L, EMB), jnp.float32),
            pltpu.VMEM((BPW,), jnp.int32),
            pltpu.VMEM((BPW, EMB), jnp.float32),
            pltpu.SemaphoreType.DMA,
            pltpu.SemaphoreType.DMA,
        ],
    )(_emb_kernel)
    return kfn(idx_flat, product, tables, product_table)


def kernel(indices, product, tables, product_table):
    idxg = indices.astype(jnp.int32) + (
        jnp.arange(F, dtype=jnp.int32) * CARD1)[:, None, None]
    idx_flat = idxg.transpose(1, 0, 2).reshape(B * F * L)
    tab_flat = jnp.concatenate([tables[f] for f in range(F)], axis=0)
    return _run(
        idx_flat,
        product.astype(jnp.int32),
        tab_flat,
        product_table,
    )


# speedup vs baseline: 3.6032x; 1.0043x over previous
import functools

import jax
import jax.numpy as jnp
from jax import lax
from jax.experimental import pallas as pl
from jax.experimental.pallas import tpu as pltpu
from jax.experimental.pallas import tpu_sc as plsc

F = 26
B = 1024
L = 200
CARD1 = 100001
PITCH = 100032
PCARD1 = 1001
PROD_OFF = F * PITCH
TROWS = PROD_OFF + 1024
EMB = 16
NSLOT = F + 1
SLAB = L * NSLOT
M = B * SLAB
NW = 32
BPW = B // NW
NS = SLAB // 128
TAIL = SLAB - NS * 128


def _emb_kernel(idx_hbm, tab_hbm, out_hbm, idx_v, buf_v, gsem, wsem):
    c = lax.axis_index("c")
    s = lax.axis_index("s")
    wid = s * 2 + c
    b0 = wid * BPW

    def do_batch(g, carry):
        b = b0 + g
        pltpu.sync_copy(idx_hbm.at[pl.ds(b * SLAB, SLAB)], idx_v)

        @pl.when(g > 0)
        def _():
            pltpu.make_async_copy(
                tab_hbm.at[pl.ds(0, SLAB)], buf_v, wsem).wait()

        def gat(j, carry2):
            pltpu.async_copy(
                tab_hbm.at[idx_v.at[pl.ds(j * 128, 128)]],
                buf_v.at[pl.ds(j * 128, 128), :],
                gsem,
            )
            return carry2

        lax.fori_loop(0, NS, gat, 0)
        pltpu.async_copy(
            tab_hbm.at[idx_v.at[pl.ds(NS * 128, TAIL)]],
            buf_v.at[pl.ds(NS * 128, TAIL), :],
            gsem,
        )

        pltpu.make_async_copy(
            tab_hbm.at[pl.ds(0, SLAB)], buf_v, gsem).wait()

        pltpu.async_copy(buf_v, out_hbm.at[pl.ds(b * SLAB, SLAB)], wsem)
        return carry

    lax.fori_loop(0, BPW, do_batch, 0)
    pltpu.make_async_copy(tab_hbm.at[pl.ds(0, SLAB)], buf_v, wsem).wait()


@jax.jit
def _run(idx_flat, tab_all):
    mesh = plsc.VectorSubcoreMesh(core_axis_name="c", subcore_axis_name="s")
    kfn = functools.partial(
        pl.kernel,
        mesh=mesh,
        compiler_params=pltpu.CompilerParams(use_tc_tiling_on_sc=False),
        out_type=jax.ShapeDtypeStruct((M, EMB), jnp.float32),
        scratch_types=[
            pltpu.VMEM((SLAB,), jnp.int32),
            pltpu.VMEM((SLAB, EMB), jnp.float32),
            pltpu.SemaphoreType.DMA,
            pltpu.SemaphoreType.DMA,
        ],
    )(_emb_kernel)
    return kfn(idx_flat, tab_all)


def kernel(indices, product, tables, product_table):
    zf = jnp.zeros((PITCH - CARD1, EMB), jnp.float32)
    zt = jnp.zeros((TROWS - PROD_OFF - PCARD1, EMB), jnp.float32)
    pieces = []
    for f in range(F):
        pieces.append(tables[f])
        pieces.append(zf)
    pieces.append(product_table)
    pieces.append(zt)
    tab_all = jnp.concatenate(pieces, axis=0)

    idxf = indices.astype(jnp.int32) + (
        jnp.arange(F, dtype=jnp.int32) * PITCH)[:, None, None]
    pidx = product.astype(jnp.int32) + PROD_OFF
    slab = jnp.concatenate(
        [
            idxf.transpose(1, 2, 0),
            jnp.broadcast_to(pidx[:, None, None], (B, L, 1)),
        ],
        axis=2,
    )
    out_flat = _run(slab.reshape(M), tab_all)
    return out_flat.reshape(B, L, NSLOT * EMB)
